# baseline (speedup 1.0000x reference)
import jax
import jax.numpy as jnp
import numpy as np
from jax import lax
import jax.experimental.pallas as pl
import jax.experimental.pallas.tpu as pltpu

N = 4608
D = 256
K = 8192
NS = 10
_TINY = np.float32(np.finfo(np.float32).tiny)

R1 = 256
R2 = 64
R3 = 64
CW = 256


def _logits_kernel(x_ref, cb_ref, sx_ref, se_ref, o_ref):
    mm = lax.dot_general(x_ref[...], cb_ref[...], (((1,), (1,)), ((), ())),
                         preferred_element_type=jnp.float32)
    d = sx_ref[...] + se_ref[...] - 2.0 * mm
    o_ref[...] = -d


def _logits(flat, codebook, sx, se):
    nl = flat.shape[0]
    return pl.pallas_call(
        _logits_kernel,
        grid=(nl // R1,),
        in_specs=[pl.BlockSpec((R1, D), lambda m: (m, 0)),
                  pl.BlockSpec((K, D), lambda m: (0, 0)),
                  pl.BlockSpec((R1, 1), lambda m: (m, 0)),
                  pl.BlockSpec((1, K), lambda m: (0, 0))],
        out_specs=pl.BlockSpec((R1, K), lambda m: (m, 0)),
        out_shape=jax.ShapeDtypeStruct((nl, K), jnp.float32),
    )(flat, codebook, sx, se)


def _threefry_bits(i):
    ks1 = jnp.uint32(42)
    ks2 = jnp.uint32(0x1BD11BDA ^ 42)

    def rotl(v, r):
        return lax.shift_left(v, jnp.uint32(r)) | lax.shift_right_logical(
            v, jnp.uint32(32 - r))

    def rounds(x0, x1, rots):
        for r in rots:
            x0 = x0 + x1
            x1 = rotl(x1, r)
            x1 = x0 ^ x1
        return x0, x1

    rot_a = (13, 15, 26, 6)
    rot_b = (17, 29, 16, 24)
    c1 = jnp.uint32(((0x1BD11BDA ^ 42) + 1) & 0xFFFFFFFF)
    c2 = jnp.uint32(2)
    c3 = jnp.uint32(45)
    c4 = jnp.uint32(((0x1BD11BDA ^ 42) + 4) & 0xFFFFFFFF)
    c5 = jnp.uint32(5)
    t = i + ks1
    x0 = t
    x1 = t ^ rotl(t, 13)
    x0, x1 = rounds(x0, x1, rot_a[1:])
    x0 = x0 + ks1; x1 = x1 + c1
    x0, x1 = rounds(x0, x1, rot_b)
    x0 = x0 + ks2; x1 = x1 + c2
    x0, x1 = rounds(x0, x1, rot_a)
    x1 = x1 + c3
    x0, x1 = rounds(x0, x1, rot_b)
    x0 = x0 + ks1; x1 = x1 + c4
    x0, x1 = rounds(x0, x1, rot_a)
    x0 = x0 + ks2; x1 = x1 + c5
    return x0 ^ x1


def _sample_kernel(off_ref, logits_ref, idx_ref):
    m = pl.program_id(0)
    s = pl.program_id(1)
    row = lax.broadcasted_iota(jnp.uint32, (R2, CW), 0)
    col = lax.broadcasted_iota(jnp.uint32, (R2, CW), 1)
    ci = lax.broadcasted_iota(jnp.int32, (R2, CW), 1)
    base = (s * N + off_ref[0] + m * R2).astype(jnp.uint32) * jnp.uint32(K)
    rowk = row * jnp.uint32(K) + col
    run_max = None
    run_arg = None
    for c in range(K // CW):
        off = c * CW
        i = rowk + (base + jnp.uint32(off))
        bits = _threefry_bits(i)
        fb = lax.shift_right_logical(bits, jnp.uint32(9)) | jnp.uint32(
            0x3F800000)
        f = lax.bitcast_convert_type(fb, jnp.float32) - jnp.float32(1.0)
        u = lax.max(_TINY, f)
        v = logits_ref[:, off:off + CW] - jnp.log(-jnp.log(u))
        cmax = jnp.max(v, axis=1, keepdims=True)
        carg = jnp.min(jnp.where(v == cmax, ci + jnp.int32(off),
                                 jnp.int32(K)), axis=1, keepdims=True)
        if c == 0:
            run_max, run_arg = cmax, carg
        else:
            upd = cmax > run_max
            run_arg = jnp.where(upd, carg, run_arg)
            run_max = jnp.maximum(run_max, cmax)
    idx_ref[pl.ds(s, 1), :, :] = run_arg[None]


def _sample(logits, off):
    nl = logits.shape[0]
    return pl.pallas_call(
        _sample_kernel,
        grid=(nl // R2, NS),
        in_specs=[pl.BlockSpec(memory_space=pltpu.SMEM),
                  pl.BlockSpec((R2, K), lambda m, s: (m, 0))],
        out_specs=pl.BlockSpec((NS, R2, 1), lambda m, s: (0, m, 0)),
        out_shape=jax.ShapeDtypeStruct((NS, nl, 1), jnp.int32),
    )(off, logits)


def _finalize_kernel(idx_ref, x_ref, cb_ref, samples_ref, qst_ref,
                     losspart_ref, histpart_ref):
    col = lax.broadcasted_iota(jnp.int32, (R3, K), 1)
    counts = jnp.zeros((R3, K), jnp.float32)
    for s in range(NS):
        counts = counts + (col == idx_ref[s, :, :]).astype(jnp.float32)
    samples_ref[...] = counts
    mmq = lax.dot_general(counts, cb_ref[...], (((1,), (0,)), ((), ())),
                          preferred_element_type=jnp.float32)
    q = mmq / jnp.float32(NS)
    xb = x_ref[...]
    qst_ref[...] = xb + (q - xb)
    dif = q - xb
    losspart_ref[0, :, :] = jnp.sum(dif * dif).reshape(1, 1)
    histpart_ref[0, :, :] = jnp.sum(counts, axis=0, keepdims=True)


def _finalize(idx, flat, codebook):
    nl = flat.shape[0]
    m3 = nl // R3
    return pl.pallas_call(
        _finalize_kernel,
        grid=(m3,),
        in_specs=[pl.BlockSpec((NS, R3, 1), lambda m: (0, m, 0)),
                  pl.BlockSpec((R3, D), lambda m: (m, 0)),
                  pl.BlockSpec((K, D), lambda m: (0, 0))],
        out_specs=[pl.BlockSpec((R3, K), lambda m: (m, 0)),
                   pl.BlockSpec((R3, D), lambda m: (m, 0)),
                   pl.BlockSpec((1, 1, 1), lambda m: (m, 0, 0)),
                   pl.BlockSpec((1, 1, K), lambda m: (m, 0, 0))],
        out_shape=[jax.ShapeDtypeStruct((nl, K), jnp.float32),
                   jax.ShapeDtypeStruct((nl, D), jnp.float32),
                   jax.ShapeDtypeStruct((m3, 1, 1), jnp.float32),
                   jax.ShapeDtypeStruct((m3, 1, K), jnp.float32)],
    )(idx, flat, codebook)


def _scalars_kernel(losspart_ref, histpart_ref, loss_ref, ppl_ref):
    total = jnp.sum(losspart_ref[...])
    e_latent = total / jnp.float32(N * D)
    loss_ref[...] = (jnp.float32(0.25) * e_latent).reshape(1, 1)
    hist = jnp.sum(histpart_ref[...], axis=0)
    avg = hist / jnp.float32(N) / jnp.float32(NS)
    ent = jnp.sum(avg * jnp.log(avg + jnp.float32(1e-10)))
    ppl_ref[...] = jnp.exp(-ent).reshape(1, 1)


def _scalars(losspart, histpart):
    m3 = losspart.shape[0]
    return pl.pallas_call(
        _scalars_kernel,
        in_specs=[pl.BlockSpec((m3, 1, 1), lambda: (0, 0, 0)),
                  pl.BlockSpec((m3, 1, K), lambda: (0, 0, 0))],
        out_specs=[pl.BlockSpec((1, 1), lambda: (0, 0)),
                   pl.BlockSpec((1, 1), lambda: (0, 0))],
        out_shape=[jax.ShapeDtypeStruct((1, 1), jnp.float32),
                   jax.ShapeDtypeStruct((1, 1), jnp.float32)],
    )(losspart, histpart)


def _run_local(flat_l, codebook, off):
    sx = jnp.sum(flat_l ** 2, axis=1, keepdims=True)
    se = jnp.sum(codebook ** 2, axis=1).reshape(1, K)
    logits = _logits(flat_l, codebook, sx, se)
    idx = _sample(logits, off)
    return _finalize(idx, flat_l, codebook)


def kernel(inputs, codebook):
    input_shape = inputs.shape
    flat = inputs.reshape(-1, D)
    devs = jax.devices()
    p = 2 if (len(devs) >= 2 and N % (2 * R1) == 0) else 1

    if p == 1:
        off = jnp.zeros((1,), jnp.int32)
        samples, qst, losspart, histpart = _run_local(flat, codebook, off)
        loss2d, ppl2d = _scalars(losspart, histpart)
    else:
        mesh = jax.sharding.Mesh(np.array(devs[:p]), ("x",))
        P_ = jax.sharding.PartitionSpec

        def f(flat_l, cb):
            off = (lax.axis_index("x") * (N // p)).astype(jnp.int32).reshape(1)
            samples, qst, losspart, histpart = _run_local(flat_l, cb, off)
            lp = lax.psum(losspart, "x")
            hp = lax.psum(histpart, "x")
            loss2d, ppl2d = _scalars(lp, hp)
            return samples, qst, loss2d, ppl2d

        samples, qst, loss2d, ppl2d = jax.shard_map(
            f, mesh=mesh,
            in_specs=(P_("x", None), P_(None, None)),
            out_specs=(P_("x", None), P_("x", None), P_(None, None),
                       P_(None, None)),
            check_vma=False,
        )(flat, codebook)

    return (qst.reshape(input_shape),
            samples.reshape(tuple(input_shape[:-1]) + (K,)),
            loss2d[0, 0],
            ppl2d[0, 0])


# speedup vs baseline: 1.0375x; 1.0375x over previous
import jax
import jax.numpy as jnp
import numpy as np
from jax import lax
import jax.experimental.pallas as pl
import jax.experimental.pallas.tpu as pltpu

N = 4608
D = 256
K = 8192
NS = 10
_TINY = np.float32(np.finfo(np.float32).tiny)

R1 = 256
R2 = 64
R3 = 64
CW = 512


def _logits_kernel(x_ref, cb_ref, sx_ref, se_ref, o_ref):
    mm = lax.dot_general(x_ref[...], cb_ref[...], (((1,), (1,)), ((), ())),
                         preferred_element_type=jnp.float32)
    d = sx_ref[...] + se_ref[...] - 2.0 * mm
    o_ref[...] = -d


def _logits(flat, codebook, sx, se):
    nl = flat.shape[0]
    return pl.pallas_call(
        _logits_kernel,
        grid=(nl // R1,),
        in_specs=[pl.BlockSpec((R1, D), lambda m: (m, 0)),
                  pl.BlockSpec((K, D), lambda m: (0, 0)),
                  pl.BlockSpec((R1, 1), lambda m: (m, 0)),
                  pl.BlockSpec((1, K), lambda m: (0, 0))],
        out_specs=pl.BlockSpec((R1, K), lambda m: (m, 0)),
        out_shape=jax.ShapeDtypeStruct((nl, K), jnp.float32),
    )(flat, codebook, sx, se)


def _threefry_bits(i):
    ks1 = jnp.uint32(42)
    ks2 = jnp.uint32(0x1BD11BDA ^ 42)

    def rotl(v, r):
        return lax.shift_left(v, jnp.uint32(r)) | lax.shift_right_logical(
            v, jnp.uint32(32 - r))

    def rounds(x0, x1, rots):
        for r in rots:
            x0 = x0 + x1
            x1 = rotl(x1, r)
            x1 = x0 ^ x1
        return x0, x1

    rot_a = (13, 15, 26, 6)
    rot_b = (17, 29, 16, 24)
    c1 = jnp.uint32(((0x1BD11BDA ^ 42) + 1) & 0xFFFFFFFF)
    c2 = jnp.uint32(2)
    c3 = jnp.uint32(45)
    c4 = jnp.uint32(((0x1BD11BDA ^ 42) + 4) & 0xFFFFFFFF)
    c5 = jnp.uint32(5)
    t = i + ks1
    x0 = t
    x1 = t ^ rotl(t, 13)
    x0, x1 = rounds(x0, x1, rot_a[1:])
    x0 = x0 + ks1; x1 = x1 + c1
    x0, x1 = rounds(x0, x1, rot_b)
    x0 = x0 + ks2; x1 = x1 + c2
    x0, x1 = rounds(x0, x1, rot_a)
    x1 = x1 + c3
    x0, x1 = rounds(x0, x1, rot_b)
    x0 = x0 + ks1; x1 = x1 + c4
    x0, x1 = rounds(x0, x1, rot_a)
    x0 = x0 + ks2; x1 = x1 + c5
    return x0 ^ x1


def _sample_kernel(off_ref, logits_ref, idx_ref):
    m = pl.program_id(0)
    s = pl.program_id(1)
    row = lax.broadcasted_iota(jnp.uint32, (R2, CW), 0)
    col = lax.broadcasted_iota(jnp.uint32, (R2, CW), 1)
    ci = lax.broadcasted_iota(jnp.int32, (R2, CW), 1)
    base = (s * N + off_ref[0] + m * R2).astype(jnp.uint32) * jnp.uint32(K)
    rowk = row * jnp.uint32(K) + col
    run_max = None
    run_arg = None
    for c in range(K // CW):
        off = c * CW
        i = rowk + (base + jnp.uint32(off))
        bits = _threefry_bits(i)
        fb = lax.shift_right_logical(bits, jnp.uint32(9)) | jnp.uint32(
            0x3F800000)
        f = lax.bitcast_convert_type(fb, jnp.float32) - jnp.float32(1.0)
        u = lax.max(_TINY, f)
        v = logits_ref[:, off:off + CW] - jnp.log(-jnp.log(u))
        cmax = jnp.max(v, axis=1, keepdims=True)
        carg = jnp.min(jnp.where(v == cmax, ci + jnp.int32(off),
                                 jnp.int32(K)), axis=1, keepdims=True)
        if c == 0:
            run_max, run_arg = cmax, carg
        else:
            upd = cmax > run_max
            run_arg = jnp.where(upd, carg, run_arg)
            run_max = jnp.maximum(run_max, cmax)
    idx_ref[pl.ds(s, 1), :, :] = run_arg[None]


def _sample(logits, off):
    nl = logits.shape[0]
    return pl.pallas_call(
        _sample_kernel,
        grid=(nl // R2, NS),
        in_specs=[pl.BlockSpec(memory_space=pltpu.SMEM),
                  pl.BlockSpec((R2, K), lambda m, s: (m, 0))],
        out_specs=pl.BlockSpec((NS, R2, 1), lambda m, s: (0, m, 0)),
        out_shape=jax.ShapeDtypeStruct((NS, nl, 1), jnp.int32),
    )(off, logits)


def _finalize_kernel(idx_ref, x_ref, cb_ref, samples_ref, qst_ref,
                     losspart_ref, histpart_ref):
    col = lax.broadcasted_iota(jnp.int32, (R3, K), 1)
    counts = jnp.zeros((R3, K), jnp.float32)
    for s in range(NS):
        counts = counts + (col == idx_ref[s, :, :]).astype(jnp.float32)
    samples_ref[...] = counts
    mmq = lax.dot_general(counts, cb_ref[...], (((1,), (0,)), ((), ())),
                          preferred_element_type=jnp.float32)
    q = mmq / jnp.float32(NS)
    xb = x_ref[...]
    qst_ref[...] = xb + (q - xb)
    dif = q - xb
    losspart_ref[0, :, :] = jnp.sum(dif * dif).reshape(1, 1)
    histpart_ref[0, :, :] = jnp.sum(counts, axis=0, keepdims=True)


def _finalize(idx, flat, codebook):
    nl = flat.shape[0]
    m3 = nl // R3
    return pl.pallas_call(
        _finalize_kernel,
        grid=(m3,),
        in_specs=[pl.BlockSpec((NS, R3, 1), lambda m: (0, m, 0)),
                  pl.BlockSpec((R3, D), lambda m: (m, 0)),
                  pl.BlockSpec((K, D), lambda m: (0, 0))],
        out_specs=[pl.BlockSpec((R3, K), lambda m: (m, 0)),
                   pl.BlockSpec((R3, D), lambda m: (m, 0)),
                   pl.BlockSpec((1, 1, 1), lambda m: (m, 0, 0)),
                   pl.BlockSpec((1, 1, K), lambda m: (m, 0, 0))],
        out_shape=[jax.ShapeDtypeStruct((nl, K), jnp.float32),
                   jax.ShapeDtypeStruct((nl, D), jnp.float32),
                   jax.ShapeDtypeStruct((m3, 1, 1), jnp.float32),
                   jax.ShapeDtypeStruct((m3, 1, K), jnp.float32)],
    )(idx, flat, codebook)


def _scalars_kernel(losspart_ref, histpart_ref, loss_ref, ppl_ref):
    total = jnp.sum(losspart_ref[...])
    e_latent = total / jnp.float32(N * D)
    loss_ref[...] = (jnp.float32(0.25) * e_latent).reshape(1, 1)
    hist = jnp.sum(histpart_ref[...], axis=0)
    avg = hist / jnp.float32(N) / jnp.float32(NS)
    ent = jnp.sum(avg * jnp.log(avg + jnp.float32(1e-10)))
    ppl_ref[...] = jnp.exp(-ent).reshape(1, 1)


def _scalars(losspart, histpart):
    m3 = losspart.shape[0]
    return pl.pallas_call(
        _scalars_kernel,
        in_specs=[pl.BlockSpec((m3, 1, 1), lambda: (0, 0, 0)),
                  pl.BlockSpec((m3, 1, K), lambda: (0, 0, 0))],
        out_specs=[pl.BlockSpec((1, 1), lambda: (0, 0)),
                   pl.BlockSpec((1, 1), lambda: (0, 0))],
        out_shape=[jax.ShapeDtypeStruct((1, 1), jnp.float32),
                   jax.ShapeDtypeStruct((1, 1), jnp.float32)],
    )(losspart, histpart)


def _run_local(flat_l, codebook, off):
    sx = jnp.sum(flat_l ** 2, axis=1, keepdims=True)
    se = jnp.sum(codebook ** 2, axis=1).reshape(1, K)
    logits = _logits(flat_l, codebook, sx, se)
    idx = _sample(logits, off)
    return _finalize(idx, flat_l, codebook)


def kernel(inputs, codebook):
    input_shape = inputs.shape
    flat = inputs.reshape(-1, D)
    devs = jax.devices()
    p = 2 if (len(devs) >= 2 and N % (2 * R1) == 0) else 1

    if p == 1:
        off = jnp.zeros((1,), jnp.int32)
        samples, qst, losspart, histpart = _run_local(flat, codebook, off)
        loss2d, ppl2d = _scalars(losspart, histpart)
    else:
        mesh = jax.sharding.Mesh(np.array(devs[:p]), ("x",))
        P_ = jax.sharding.PartitionSpec

        def f(flat_l, cb):
            off = (lax.axis_index("x") * (N // p)).astype(jnp.int32).reshape(1)
            samples, qst, losspart, histpart = _run_local(flat_l, cb, off)
            lp = lax.psum(losspart, "x")
            hp = lax.psum(histpart, "x")
            loss2d, ppl2d = _scalars(lp, hp)
            return samples, qst, loss2d, ppl2d

        samples, qst, loss2d, ppl2d = jax.shard_map(
            f, mesh=mesh,
            in_specs=(P_("x", None), P_(None, None)),
            out_specs=(P_("x", None), P_("x", None), P_(None, None),
                       P_(None, None)),
            check_vma=False,
        )(flat, codebook)

    return (qst.reshape(input_shape),
            samples.reshape(tuple(input_shape[:-1]) + (K,)),
            loss2d[0, 0],
            ppl2d[0, 0])
